# baseline (speedup 1.0000x reference)
import functools

import jax
import jax.numpy as jnp
from jax.experimental import pallas as pl
from jax.experimental.pallas import tpu as pltpu

_BATCH = 32
_VOCAB = 1_000_000
_CHUNK = 8192

_K0 = 0
_K1 = 42
_KS2 = 0x1BD11BDA ^ _K0 ^ _K1
_TINY = float(jnp.finfo(jnp.float32).tiny)

_ROT = ((13, 15, 26, 6), (17, 29, 16, 24))


def _rotl(x, r):
    return (x << jnp.uint32(r)) | (x >> jnp.uint32(32 - r))


def _threefry_bits(flat):
    ks = (jnp.uint32(_K0), jnp.uint32(_K1), jnp.uint32(_KS2))
    x0 = jnp.full_like(flat, ks[0])
    x1 = flat + ks[1]
    for i in range(5):
        for r in _ROT[i % 2]:
            x0 = x0 + x1
            x1 = _rotl(x1, r) ^ x0
        x0 = x0 + ks[(i + 1) % 3]
        x1 = x1 + ks[(i + 2) % 3] + jnp.uint32(i + 1)
    return x0 ^ x1


def _gumbel_from_bits(bits):
    fb = (bits >> jnp.uint32(9)) | jnp.uint32(0x3F800000)
    f = jax.lax.bitcast_convert_type(fb, jnp.float32) - jnp.float32(1.0)
    tiny = jnp.float32(_TINY)
    u = jnp.maximum(tiny, f * (jnp.float32(1.0) - tiny) + tiny)
    return -jnp.log(-jnp.log(u))


def _sample_kernel(x_ref, ov_ref, oi_ref, acc_val, acc_idx, *, steps_per_core,
                   nblocks):
    c = pl.program_id(0)
    j = pl.program_id(1)

    @pl.when(j == 0)
    def _init():
        acc_val[...] = jnp.full((_BATCH, _CHUNK), -jnp.inf, jnp.float32)
        acc_idx[...] = jnp.zeros((_BATCH, _CHUNK), jnp.int32)

    row = jax.lax.broadcasted_iota(jnp.uint32, (_BATCH, _CHUNK), 0)
    col = jax.lax.broadcasted_iota(jnp.uint32, (_BATCH, _CHUNK), 1)
    blk = jnp.minimum(c * steps_per_core + j, nblocks - 1)
    base = blk.astype(jnp.uint32) * jnp.uint32(_CHUNK)
    gcol = col + base
    flat = row * jnp.uint32(_VOCAB) + gcol

    g = _gumbel_from_bits(_threefry_bits(flat))
    val = x_ref[...] + g
    val = jnp.where(gcol.astype(jnp.int32) < _VOCAB, val, -jnp.inf)

    take = val > acc_val[...]
    acc_val[...] = jnp.where(take, val, acc_val[...])
    acc_idx[...] = jnp.where(take, gcol.astype(jnp.int32), acc_idx[...])

    @pl.when(j == steps_per_core - 1)
    def _finish():
        av = acc_val[...]
        m = jnp.max(av, axis=1, keepdims=True)
        cand = jnp.where(av == m, acc_idx[...], jnp.int32(0x7FFFFFFF))
        ov_ref[...] = m.reshape(1, _BATCH, 1)
        oi_ref[...] = jnp.min(cand, axis=1, keepdims=True).reshape(1, _BATCH, 1)


def _merge_kernel(pv_ref, pi_ref, o_ref):
    v = pv_ref[...]
    i = pi_ref[...]
    take1 = v[1] > v[0]
    o_ref[...] = jnp.where(take1, i[1], i[0])


def kernel(logits):
    nblocks = (_VOCAB + _CHUNK - 1) // _CHUNK
    steps = (nblocks + 1) // 2
    pv, pi = pl.pallas_call(
        functools.partial(_sample_kernel, steps_per_core=steps, nblocks=nblocks),
        grid=(2, steps),
        in_specs=[
            pl.BlockSpec(
                (_BATCH, _CHUNK),
                lambda c, j, s=steps, n=nblocks: (0, jnp.minimum(c * s + j, n - 1)),
            ),
        ],
        out_specs=[
            pl.BlockSpec((1, _BATCH, 1), lambda c, j: (c, 0, 0)),
            pl.BlockSpec((1, _BATCH, 1), lambda c, j: (c, 0, 0)),
        ],
        out_shape=[
            jax.ShapeDtypeStruct((2, _BATCH, 1), jnp.float32),
            jax.ShapeDtypeStruct((2, _BATCH, 1), jnp.int32),
        ],
        scratch_shapes=[
            pltpu.VMEM((_BATCH, _CHUNK), jnp.float32),
            pltpu.VMEM((_BATCH, _CHUNK), jnp.int32),
        ],
        compiler_params=pltpu.CompilerParams(
            dimension_semantics=("parallel", "arbitrary"),
        ),
    )(logits)
    return pl.pallas_call(
        _merge_kernel,
        out_shape=jax.ShapeDtypeStruct((_BATCH, 1), jnp.int32),
    )(pv, pi)
